# baseline (speedup 1.0000x reference)
import functools

import jax
import jax.numpy as jnp
from jax import lax
from jax.experimental import pallas as pl
from jax.experimental.pallas import tpu as pltpu
from jax.experimental.pallas import tpu_sc as plsc

_N = 10000
_E = 320000
_D = 128
_G = 64

_NC = 2
_NS = 16
_NW = _NC * _NS
_CH = 128
_CPW = 80
_EPAD = _NW * _CPW * _CH
_NBUF = 2
_TRASH = 128
_AROWS = _N + _TRASH
_CPS = 40
_NSTAGE = _CPW // _CPS
_ZR = 40
_ZBLOCKS = _N // _ZR
_ZITERS = -(-_ZBLOCKS // _NS)
_WR = 1000
_WBLOCKS = _N // _WR

_BM = 5000
_NB = _N // _BM


def _sc_segment_sum(x, src3, dst3):
    mesh = plsc.VectorSubcoreMesh(core_axis_name="core", subcore_axis_name="subcore")

    @functools.partial(
        pl.kernel,
        out_type=jax.ShapeDtypeStruct((_NC, _N, _D), jnp.float32),
        mesh=mesh,
        scratch_types=[
            pltpu.VMEM((_CPS, _CH), jnp.int32),
            pltpu.VMEM((_CPS, _CH), jnp.int32),
            pltpu.VMEM((_NBUF, _CH, _D), jnp.float32),
            pltpu.VMEM((_ZR, _D), jnp.float32),
            pltpu.VMEM_SHARED((_AROWS, _D), jnp.float32),
            pltpu.SemaphoreType.DMA,
            pltpu.SemaphoreType.DMA,
            pltpu.SemaphoreType.DMA,
            pltpu.SemaphoreType.DMA,
        ],
    )
    def agg(x_hbm, src_hbm, dst_hbm, out_hbm, src_v, dst_v, rows_v,
            zero_v, acc_sh, sem0, sem1, sem2, sem3):
        cid = lax.axis_index("core")
        sid = lax.axis_index("subcore")
        wid = cid * _NS + sid

        @pl.loop(0, _ZR)
        def _(r):
            @pl.loop(0, _D, step=16)
            def _(c):
                zero_v[r, pl.ds(c, 16)] = jnp.zeros((16,), jnp.float32)

        @pl.loop(0, _ZITERS)
        def _(j):
            blk = j * _NS + sid

            @pl.when(blk < _ZBLOCKS)
            def _():
                pltpu.sync_copy(zero_v, acc_sh.at[pl.ds(blk * _ZR, _ZR)])

        plsc.subcore_barrier()

        @pl.loop(0, _NSTAGE)
        def _(s):
            pltpu.sync_copy(src_hbm.at[wid, pl.ds(s * _CPS, _CPS)], src_v)
            pltpu.sync_copy(dst_hbm.at[wid, pl.ds(s * _CPS, _CPS)], dst_v)
            pltpu.async_copy(x_hbm.at[src_v.at[0]], rows_v.at[0], sem0)
            pltpu.async_copy(x_hbm.at[src_v.at[1]], rows_v.at[1], sem1)

            @pl.loop(0, _CPS - 2, step=2)
            def _(j):
                pltpu.make_async_copy(x_hbm.at[src_v.at[j]], rows_v.at[0],
                                      sem0).wait()
                pltpu.async_copy(rows_v.at[0], acc_sh.at[dst_v.at[j]], sem2,
                                 add=True)
                pltpu.make_async_copy(x_hbm.at[src_v.at[j + 1]], rows_v.at[1],
                                      sem1).wait()
                pltpu.async_copy(rows_v.at[1], acc_sh.at[dst_v.at[j + 1]],
                                 sem3, add=True)
                pltpu.make_async_copy(rows_v.at[0], acc_sh.at[dst_v.at[j]],
                                      sem2).wait()
                pltpu.async_copy(x_hbm.at[src_v.at[j + 2]], rows_v.at[0], sem0)
                pltpu.make_async_copy(rows_v.at[1], acc_sh.at[dst_v.at[j + 1]],
                                      sem3).wait()
                pltpu.async_copy(x_hbm.at[src_v.at[j + 3]], rows_v.at[1], sem1)

            pltpu.make_async_copy(x_hbm.at[src_v.at[_CPS - 2]], rows_v.at[0],
                                  sem0).wait()
            pltpu.sync_copy(rows_v.at[0], acc_sh.at[dst_v.at[_CPS - 2]],
                            add=True)
            pltpu.make_async_copy(x_hbm.at[src_v.at[_CPS - 1]], rows_v.at[1],
                                  sem1).wait()
            pltpu.sync_copy(rows_v.at[1], acc_sh.at[dst_v.at[_CPS - 1]],
                            add=True)

        plsc.subcore_barrier()

        @pl.when(sid < _WBLOCKS)
        def _():
            pltpu.sync_copy(acc_sh.at[pl.ds(sid * _WR, _WR)],
                            out_hbm.at[cid, pl.ds(sid * _WR, _WR)])

    return agg(x, src3, dst3)


def _mlp(x, agg, W1, b1, W2, b2):
    def body(x_ref, a0_ref, a1_ref, w1_ref, b1_ref, w2_ref, b2_ref, o_ref):
        h = x_ref[...] + a0_ref[0] + a1_ref[0]
        h = jnp.dot(h.astype(jnp.bfloat16), w1_ref[...].astype(jnp.bfloat16),
                    preferred_element_type=jnp.float32)
        h = jnp.maximum(h + b1_ref[...], 0.0)
        h = jnp.dot(h.astype(jnp.bfloat16), w2_ref[...].astype(jnp.bfloat16),
                    preferred_element_type=jnp.float32)
        o_ref[...] = jnp.maximum(h + b2_ref[...], 0.0)

    return pl.pallas_call(
        body,
        grid=(_NB,),
        in_specs=[
            pl.BlockSpec((_BM, _D), lambda i: (i, 0)),
            pl.BlockSpec((1, _BM, _D), lambda i: (0, i, 0)),
            pl.BlockSpec((1, _BM, _D), lambda i: (1, i, 0)),
            pl.BlockSpec((_D, _D), lambda i: (0, 0)),
            pl.BlockSpec((1, _D), lambda i: (0, 0)),
            pl.BlockSpec((_D, _D), lambda i: (0, 0)),
            pl.BlockSpec((1, _D), lambda i: (0, 0)),
        ],
        out_specs=pl.BlockSpec((_BM, _D), lambda i: (i, 0)),
        out_shape=jax.ShapeDtypeStruct((_N, _D), jnp.float32),
    )(x, agg, agg, W1, b1.reshape(1, _D), W2, b2.reshape(1, _D))


def _mlp_pool(h, agg, W1, b1, W2, b2, batch3, Wf, bf):
    def body(h_ref, a0_ref, a1_ref, w1_ref, b1_ref, w2_ref, b2_ref,
             batch_ref, wf_ref, bf_ref, o_ref, sums, counts):
        i = pl.program_id(0)

        @pl.when(i == 0)
        def _():
            sums[...] = jnp.zeros_like(sums)
            counts[...] = jnp.zeros_like(counts)

        h2 = h_ref[...] + a0_ref[0] + a1_ref[0]
        h2 = jnp.dot(h2.astype(jnp.bfloat16), w1_ref[...].astype(jnp.bfloat16),
                     preferred_element_type=jnp.float32)
        h2 = jnp.maximum(h2 + b1_ref[...], 0.0)
        h2 = jnp.dot(h2.astype(jnp.bfloat16), w2_ref[...].astype(jnp.bfloat16),
                     preferred_element_type=jnp.float32)
        h2 = jnp.maximum(h2 + b2_ref[...], 0.0)

        b = batch_ref[0, 0, :]
        gid = lax.broadcasted_iota(jnp.int32, (_G, _BM), 0)
        mask = (b[None, :] == gid).astype(jnp.float32)
        sums[...] += jnp.dot(mask, h2, preferred_element_type=jnp.float32,
                     precision=lax.Precision.HIGHEST)
        counts[...] += jnp.sum(mask, axis=1, keepdims=True)

        @pl.when(i == _NB - 1)
        def _():
            pooled = sums[...] / jnp.maximum(counts[...], 1.0)
            o_ref[...] = (
                jnp.dot(pooled.astype(jnp.bfloat16),
                        wf_ref[...].astype(jnp.bfloat16),
                        preferred_element_type=jnp.float32)
                + bf_ref[...])

    return pl.pallas_call(
        body,
        grid=(_NB,),
        in_specs=[
            pl.BlockSpec((_BM, _D), lambda i: (i, 0)),
            pl.BlockSpec((1, _BM, _D), lambda i: (0, i, 0)),
            pl.BlockSpec((1, _BM, _D), lambda i: (1, i, 0)),
            pl.BlockSpec((_D, _D), lambda i: (0, 0)),
            pl.BlockSpec((1, _D), lambda i: (0, 0)),
            pl.BlockSpec((_D, _D), lambda i: (0, 0)),
            pl.BlockSpec((1, _D), lambda i: (0, 0)),
            pl.BlockSpec((1, 1, _BM), lambda i: (i, 0, 0)),
            pl.BlockSpec((_D, 1), lambda i: (0, 0)),
            pl.BlockSpec((1, 1), lambda i: (0, 0)),
        ],
        out_specs=pl.BlockSpec((_G, 1), lambda i: (0, 0)),
        out_shape=jax.ShapeDtypeStruct((_G, 1), jnp.float32),
        scratch_shapes=[
            pltpu.VMEM((_G, _D), jnp.float32),
            pltpu.VMEM((_G, 1), jnp.float32),
        ],
    )(h, agg, agg, W1, b1.reshape(1, _D), W2, b2.reshape(1, _D),
      batch3, Wf, bf.reshape(1, 1))


def kernel(x, edge_index, batch, W1_0, b1_0, W2_0, b2_0, W1_1, b1_1,
           W2_1, b2_1, Wf, bf):
    npad = _EPAD - _E
    src3 = jnp.concatenate(
        [edge_index[0],
         jnp.arange(npad, dtype=jnp.int32) % jnp.int32(_N)]
    ).reshape(_NW, _CPW, _CH)
    dst3 = jnp.concatenate(
        [edge_index[1],
         _N + (jnp.arange(npad, dtype=jnp.int32) % _TRASH)]
    ).reshape(_NW, _CPW, _CH)
    agg0 = _sc_segment_sum(x, src3, dst3)
    h1 = _mlp(x, agg0, W1_0, b1_0, W2_0, b2_0)
    agg1 = _sc_segment_sum(h1, src3, dst3)
    batch3 = batch.reshape(_NB, 1, _BM)
    out = _mlp_pool(h1, agg1, W1_1, b1_1, W2_1, b2_1, batch3, Wf, bf)
    return out[:, 0]


# speedup vs baseline: 1.2784x; 1.2784x over previous
import functools

import jax
import jax.numpy as jnp
from jax import lax
from jax.experimental import pallas as pl
from jax.experimental.pallas import tpu as pltpu
from jax.experimental.pallas import tpu_sc as plsc

_N = 10000
_E = 320000
_D = 128
_G = 64

_NC = 2
_NS = 16
_NW = _NC * _NS
_CH = 128
_CPW = 80
_EPAD = _NW * _CPW * _CH
_NBUF = 2
_TRASH = 128
_AROWS = _N + _TRASH
_CPS = 40
_NSTAGE = _CPW // _CPS
_ZR = 40
_ZBLOCKS = _N // _ZR
_ZITERS = -(-_ZBLOCKS // _NS)
_WR = 1000
_WBLOCKS = _N // _WR

_BM = 5000
_NB = _N // _BM


def _sc_segment_sum(x, src3, dst3):
    mesh = plsc.VectorSubcoreMesh(core_axis_name="core", subcore_axis_name="subcore")

    @functools.partial(
        pl.kernel,
        out_type=jax.ShapeDtypeStruct((_NC, _N, _D), jnp.float32),
        mesh=mesh,
        scratch_types=[
            pltpu.VMEM((_CPS, _CH), jnp.int32),
            pltpu.VMEM((_CPS, _CH), jnp.int32),
            pltpu.VMEM((_NBUF, _CH, _D), jnp.float32),
            pltpu.VMEM((_ZR, _D), jnp.float32),
            pltpu.VMEM_SHARED((_AROWS, _D), jnp.float32),
            pltpu.SemaphoreType.DMA,
            pltpu.SemaphoreType.DMA,
            pltpu.SemaphoreType.DMA,
            pltpu.SemaphoreType.DMA,
        ],
    )
    def agg(x_hbm, src_hbm, dst_hbm, out_hbm, src_v, dst_v, rows_v,
            zero_v, acc_sh, sem0, sem1, sem2, sem3):
        cid = lax.axis_index("core")
        sid = lax.axis_index("subcore")
        wid = cid * _NS + sid

        @pl.loop(0, _ZR)
        def _(r):
            @pl.loop(0, _D, step=16)
            def _(c):
                zero_v[r, pl.ds(c, 16)] = jnp.zeros((16,), jnp.float32)

        @pl.loop(0, _ZITERS)
        def _(j):
            blk = j * _NS + sid

            @pl.when(blk < _ZBLOCKS)
            def _():
                pltpu.sync_copy(zero_v, acc_sh.at[pl.ds(blk * _ZR, _ZR)])

        plsc.subcore_barrier()

        @pl.loop(0, _NSTAGE)
        def _(s):
            pltpu.sync_copy(src_hbm.at[wid, pl.ds(s * _CPS, _CPS)], src_v)
            pltpu.sync_copy(dst_hbm.at[wid, pl.ds(s * _CPS, _CPS)], dst_v)
            pltpu.async_copy(x_hbm.at[src_v.at[0]], rows_v.at[0], sem0)

            @pl.loop(0, _CPS, step=2)
            def _(j):
                pltpu.async_copy(x_hbm.at[src_v.at[j + 1]], rows_v.at[1], sem1)
                pltpu.make_async_copy(x_hbm.at[src_v.at[j]], rows_v.at[0],
                                      sem0).wait()
                pltpu.sync_copy(rows_v.at[0], acc_sh.at[dst_v.at[j]], add=True)

                @pl.when(j + 2 < _CPS)
                def _():
                    pltpu.async_copy(x_hbm.at[src_v.at[j + 2]], rows_v.at[0],
                                     sem0)

                pltpu.make_async_copy(x_hbm.at[src_v.at[j + 1]], rows_v.at[1],
                                      sem1).wait()
                pltpu.sync_copy(rows_v.at[1], acc_sh.at[dst_v.at[j + 1]],
                                add=True)

        plsc.subcore_barrier()

        @pl.when(sid < _WBLOCKS)
        def _():
            pltpu.sync_copy(acc_sh.at[pl.ds(sid * _WR, _WR)],
                            out_hbm.at[cid, pl.ds(sid * _WR, _WR)])

    return agg(x, src3, dst3)


def _mlp(x, agg, W1, b1, W2, b2):
    def body(x_ref, a0_ref, a1_ref, w1_ref, b1_ref, w2_ref, b2_ref, o_ref):
        h = x_ref[...] + a0_ref[0] + a1_ref[0]
        h = jnp.dot(h.astype(jnp.bfloat16), w1_ref[...].astype(jnp.bfloat16),
                    preferred_element_type=jnp.float32)
        h = jnp.maximum(h + b1_ref[...], 0.0)
        h = jnp.dot(h.astype(jnp.bfloat16), w2_ref[...].astype(jnp.bfloat16),
                    preferred_element_type=jnp.float32)
        o_ref[...] = jnp.maximum(h + b2_ref[...], 0.0)

    return pl.pallas_call(
        body,
        grid=(_NB,),
        in_specs=[
            pl.BlockSpec((_BM, _D), lambda i: (i, 0)),
            pl.BlockSpec((1, _BM, _D), lambda i: (0, i, 0)),
            pl.BlockSpec((1, _BM, _D), lambda i: (1, i, 0)),
            pl.BlockSpec((_D, _D), lambda i: (0, 0)),
            pl.BlockSpec((1, _D), lambda i: (0, 0)),
            pl.BlockSpec((_D, _D), lambda i: (0, 0)),
            pl.BlockSpec((1, _D), lambda i: (0, 0)),
        ],
        out_specs=pl.BlockSpec((_BM, _D), lambda i: (i, 0)),
        out_shape=jax.ShapeDtypeStruct((_N, _D), jnp.float32),
    )(x, agg, agg, W1, b1.reshape(1, _D), W2, b2.reshape(1, _D))


def _mlp_pool(h, agg, W1, b1, W2, b2, batch3, Wf, bf):
    def body(h_ref, a0_ref, a1_ref, w1_ref, b1_ref, w2_ref, b2_ref,
             batch_ref, wf_ref, bf_ref, o_ref, sums, counts):
        i = pl.program_id(0)

        @pl.when(i == 0)
        def _():
            sums[...] = jnp.zeros_like(sums)
            counts[...] = jnp.zeros_like(counts)

        h2 = h_ref[...] + a0_ref[0] + a1_ref[0]
        h2 = jnp.dot(h2.astype(jnp.bfloat16), w1_ref[...].astype(jnp.bfloat16),
                     preferred_element_type=jnp.float32)
        h2 = jnp.maximum(h2 + b1_ref[...], 0.0)
        h2 = jnp.dot(h2.astype(jnp.bfloat16), w2_ref[...].astype(jnp.bfloat16),
                     preferred_element_type=jnp.float32)
        h2 = jnp.maximum(h2 + b2_ref[...], 0.0)

        b = batch_ref[0, 0, :]
        gid = lax.broadcasted_iota(jnp.int32, (_G, _BM), 0)
        mask = (b[None, :] == gid).astype(jnp.float32)
        sums[...] += jnp.dot(mask, h2, preferred_element_type=jnp.float32,
                     precision=lax.Precision.HIGHEST)
        counts[...] += jnp.sum(mask, axis=1, keepdims=True)

        @pl.when(i == _NB - 1)
        def _():
            pooled = sums[...] / jnp.maximum(counts[...], 1.0)
            o_ref[...] = (
                jnp.dot(pooled.astype(jnp.bfloat16),
                        wf_ref[...].astype(jnp.bfloat16),
                        preferred_element_type=jnp.float32)
                + bf_ref[...])

    return pl.pallas_call(
        body,
        grid=(_NB,),
        in_specs=[
            pl.BlockSpec((_BM, _D), lambda i: (i, 0)),
            pl.BlockSpec((1, _BM, _D), lambda i: (0, i, 0)),
            pl.BlockSpec((1, _BM, _D), lambda i: (1, i, 0)),
            pl.BlockSpec((_D, _D), lambda i: (0, 0)),
            pl.BlockSpec((1, _D), lambda i: (0, 0)),
            pl.BlockSpec((_D, _D), lambda i: (0, 0)),
            pl.BlockSpec((1, _D), lambda i: (0, 0)),
            pl.BlockSpec((1, 1, _BM), lambda i: (i, 0, 0)),
            pl.BlockSpec((_D, 1), lambda i: (0, 0)),
            pl.BlockSpec((1, 1), lambda i: (0, 0)),
        ],
        out_specs=pl.BlockSpec((_G, 1), lambda i: (0, 0)),
        out_shape=jax.ShapeDtypeStruct((_G, 1), jnp.float32),
        scratch_shapes=[
            pltpu.VMEM((_G, _D), jnp.float32),
            pltpu.VMEM((_G, 1), jnp.float32),
        ],
    )(h, agg, agg, W1, b1.reshape(1, _D), W2, b2.reshape(1, _D),
      batch3, Wf, bf.reshape(1, 1))


def kernel(x, edge_index, batch, W1_0, b1_0, W2_0, b2_0, W1_1, b1_1,
           W2_1, b2_1, Wf, bf):
    npad = _EPAD - _E
    src3 = jnp.concatenate(
        [edge_index[0],
         jnp.arange(npad, dtype=jnp.int32) % jnp.int32(_N)]
    ).reshape(_NW, _CPW, _CH)
    dst3 = jnp.concatenate(
        [edge_index[1],
         _N + (jnp.arange(npad, dtype=jnp.int32) % _TRASH)]
    ).reshape(_NW, _CPW, _CH)
    agg0 = _sc_segment_sum(x, src3, dst3)
    h1 = _mlp(x, agg0, W1_0, b1_0, W2_0, b2_0)
    agg1 = _sc_segment_sum(h1, src3, dst3)
    batch3 = batch.reshape(_NB, 1, _BM)
    out = _mlp_pool(h1, agg1, W1_1, b1_1, W2_1, b2_1, batch3, Wf, bf)
    return out[:, 0]


# speedup vs baseline: 1.2964x; 1.0141x over previous
import functools

import jax
import jax.numpy as jnp
from jax import lax
from jax.experimental import pallas as pl
from jax.experimental.pallas import tpu as pltpu
from jax.experimental.pallas import tpu_sc as plsc

_N = 10000
_E = 320000
_D = 128
_G = 64

_NC = 2
_NS = 16
_NW = _NC * _NS
_CH = 128
_CPW = 80
_EPAD = _NW * _CPW * _CH
_NBUF = 2
_TRASH = 128
_AROWS = _N + _TRASH
_CPS = 40
_NSTAGE = _CPW // _CPS
_ZR = 40
_ZBLOCKS = _N // _ZR
_ZITERS = -(-_ZBLOCKS // _NS)
_WR = 1000
_WBLOCKS = _N // _WR

_BM = 5000
_NB = _N // _BM


def _sc_segment_sum(x, src3, dst3):
    mesh = plsc.VectorSubcoreMesh(core_axis_name="core", subcore_axis_name="subcore")

    @functools.partial(
        pl.kernel,
        out_type=jax.ShapeDtypeStruct((_NC, _N, _D), jnp.float32),
        mesh=mesh,
        scratch_types=[
            pltpu.VMEM((_CPS, _CH), jnp.int32),
            pltpu.VMEM((_CPS, _CH), jnp.int32),
            pltpu.VMEM((_NBUF, _CH, _D), jnp.float32),
            pltpu.VMEM((_ZR, _D), jnp.float32),
            pltpu.VMEM_SHARED((_AROWS, _D), jnp.float32),
            pltpu.SemaphoreType.DMA,
            pltpu.SemaphoreType.DMA,
            pltpu.SemaphoreType.DMA,
            pltpu.SemaphoreType.DMA,
        ],
    )
    def agg(x_hbm, src_hbm, dst_hbm, out_hbm, src_v, dst_v, rows_v,
            zero_v, acc_sh, sem0, sem1, sem2, sem3):
        cid = lax.axis_index("core")
        sid = lax.axis_index("subcore")
        wid = cid * _NS + sid

        pltpu.sync_copy(src_hbm.at[wid, pl.ds(0, _CPS)], src_v)
        pltpu.sync_copy(dst_hbm.at[wid, pl.ds(0, _CPS)], dst_v)
        pltpu.async_copy(x_hbm.at[src_v.at[0]], rows_v.at[0], sem0)

        @pl.loop(0, _ZR)
        def _(r):
            @pl.loop(0, _D, step=16)
            def _(c):
                zero_v[r, pl.ds(c, 16)] = jnp.zeros((16,), jnp.float32)

        @pl.loop(0, _ZITERS)
        def _(j):
            blk = j * _NS + sid

            @pl.when(blk < _ZBLOCKS)
            def _():
                pltpu.sync_copy(zero_v, acc_sh.at[pl.ds(blk * _ZR, _ZR)])

        plsc.subcore_barrier()

        def run_stage(s, prologue):
            if prologue:
                pltpu.sync_copy(src_hbm.at[wid, pl.ds(s * _CPS, _CPS)], src_v)
                pltpu.sync_copy(dst_hbm.at[wid, pl.ds(s * _CPS, _CPS)], dst_v)
                pltpu.async_copy(x_hbm.at[src_v.at[0]], rows_v.at[0], sem0)

            @pl.loop(0, _CPS, step=2)
            def _(j):
                pltpu.async_copy(x_hbm.at[src_v.at[j + 1]], rows_v.at[1], sem1)
                pltpu.make_async_copy(x_hbm.at[src_v.at[j]], rows_v.at[0],
                                      sem0).wait()
                pltpu.sync_copy(rows_v.at[0], acc_sh.at[dst_v.at[j]], add=True)

                @pl.when(j + 2 < _CPS)
                def _():
                    pltpu.async_copy(x_hbm.at[src_v.at[j + 2]], rows_v.at[0],
                                     sem0)

                pltpu.make_async_copy(x_hbm.at[src_v.at[j + 1]], rows_v.at[1],
                                      sem1).wait()
                pltpu.sync_copy(rows_v.at[1], acc_sh.at[dst_v.at[j + 1]],
                                add=True)

        for s in range(_NSTAGE):
            run_stage(s, prologue=(s > 0))

        plsc.subcore_barrier()

        @pl.when(sid < _WBLOCKS)
        def _():
            pltpu.sync_copy(acc_sh.at[pl.ds(sid * _WR, _WR)],
                            out_hbm.at[cid, pl.ds(sid * _WR, _WR)])

    return agg(x, src3, dst3)


def _mlp(x, agg, W1, b1, W2, b2):
    def body(x_ref, a0_ref, a1_ref, w1_ref, b1_ref, w2_ref, b2_ref, o_ref):
        h = x_ref[...] + a0_ref[0] + a1_ref[0]
        h = jnp.dot(h.astype(jnp.bfloat16), w1_ref[...].astype(jnp.bfloat16),
                    preferred_element_type=jnp.float32)
        h = jnp.maximum(h + b1_ref[...], 0.0)
        h = jnp.dot(h.astype(jnp.bfloat16), w2_ref[...].astype(jnp.bfloat16),
                    preferred_element_type=jnp.float32)
        o_ref[...] = jnp.maximum(h + b2_ref[...], 0.0)

    return pl.pallas_call(
        body,
        grid=(_NB,),
        in_specs=[
            pl.BlockSpec((_BM, _D), lambda i: (i, 0)),
            pl.BlockSpec((1, _BM, _D), lambda i: (0, i, 0)),
            pl.BlockSpec((1, _BM, _D), lambda i: (1, i, 0)),
            pl.BlockSpec((_D, _D), lambda i: (0, 0)),
            pl.BlockSpec((1, _D), lambda i: (0, 0)),
            pl.BlockSpec((_D, _D), lambda i: (0, 0)),
            pl.BlockSpec((1, _D), lambda i: (0, 0)),
        ],
        out_specs=pl.BlockSpec((_BM, _D), lambda i: (i, 0)),
        out_shape=jax.ShapeDtypeStruct((_N, _D), jnp.float32),
    )(x, agg, agg, W1, b1.reshape(1, _D), W2, b2.reshape(1, _D))


def _mlp_pool(h, agg, W1, b1, W2, b2, batch3, Wf, bf):
    def body(h_ref, a0_ref, a1_ref, w1_ref, b1_ref, w2_ref, b2_ref,
             batch_ref, wf_ref, bf_ref, o_ref, sums, counts):
        i = pl.program_id(0)

        @pl.when(i == 0)
        def _():
            sums[...] = jnp.zeros_like(sums)
            counts[...] = jnp.zeros_like(counts)

        h2 = h_ref[...] + a0_ref[0] + a1_ref[0]
        h2 = jnp.dot(h2.astype(jnp.bfloat16), w1_ref[...].astype(jnp.bfloat16),
                     preferred_element_type=jnp.float32)
        h2 = jnp.maximum(h2 + b1_ref[...], 0.0)
        h2 = jnp.dot(h2.astype(jnp.bfloat16), w2_ref[...].astype(jnp.bfloat16),
                     preferred_element_type=jnp.float32)
        h2 = jnp.maximum(h2 + b2_ref[...], 0.0)

        b = batch_ref[0, 0, :]
        gid = lax.broadcasted_iota(jnp.int32, (_G, _BM), 0)
        mask = (b[None, :] == gid).astype(jnp.float32)
        sums[...] += jnp.dot(mask, h2, preferred_element_type=jnp.float32,
                     precision=lax.Precision.HIGHEST)
        counts[...] += jnp.sum(mask, axis=1, keepdims=True)

        @pl.when(i == _NB - 1)
        def _():
            pooled = sums[...] / jnp.maximum(counts[...], 1.0)
            o_ref[...] = (
                jnp.dot(pooled.astype(jnp.bfloat16),
                        wf_ref[...].astype(jnp.bfloat16),
                        preferred_element_type=jnp.float32)
                + bf_ref[...])

    return pl.pallas_call(
        body,
        grid=(_NB,),
        in_specs=[
            pl.BlockSpec((_BM, _D), lambda i: (i, 0)),
            pl.BlockSpec((1, _BM, _D), lambda i: (0, i, 0)),
            pl.BlockSpec((1, _BM, _D), lambda i: (1, i, 0)),
            pl.BlockSpec((_D, _D), lambda i: (0, 0)),
            pl.BlockSpec((1, _D), lambda i: (0, 0)),
            pl.BlockSpec((_D, _D), lambda i: (0, 0)),
            pl.BlockSpec((1, _D), lambda i: (0, 0)),
            pl.BlockSpec((1, 1, _BM), lambda i: (i, 0, 0)),
            pl.BlockSpec((_D, 1), lambda i: (0, 0)),
            pl.BlockSpec((1, 1), lambda i: (0, 0)),
        ],
        out_specs=pl.BlockSpec((_G, 1), lambda i: (0, 0)),
        out_shape=jax.ShapeDtypeStruct((_G, 1), jnp.float32),
        scratch_shapes=[
            pltpu.VMEM((_G, _D), jnp.float32),
            pltpu.VMEM((_G, 1), jnp.float32),
        ],
    )(h, agg, agg, W1, b1.reshape(1, _D), W2, b2.reshape(1, _D),
      batch3, Wf, bf.reshape(1, 1))


def kernel(x, edge_index, batch, W1_0, b1_0, W2_0, b2_0, W1_1, b1_1,
           W2_1, b2_1, Wf, bf):
    npad = _EPAD - _E
    src3 = jnp.concatenate(
        [edge_index[0],
         jnp.arange(npad, dtype=jnp.int32) % jnp.int32(_N)]
    ).reshape(_NW, _CPW, _CH)
    dst3 = jnp.concatenate(
        [edge_index[1],
         _N + (jnp.arange(npad, dtype=jnp.int32) % _TRASH)]
    ).reshape(_NW, _CPW, _CH)
    agg0 = _sc_segment_sum(x, src3, dst3)
    h1 = _mlp(x, agg0, W1_0, b1_0, W2_0, b2_0)
    agg1 = _sc_segment_sum(h1, src3, dst3)
    batch3 = batch.reshape(_NB, 1, _BM)
    out = _mlp_pool(h1, agg1, W1_1, b1_1, W2_1, b2_1, batch3, Wf, bf)
    return out[:, 0]
